# baseline (speedup 1.0000x reference)
import functools

import jax
import jax.numpy as jnp
from jax import lax
from jax.experimental import pallas as pl
from jax.experimental.pallas import tpu as pltpu
from jax.experimental.pallas import tpu_sc as plsc

VOCAB = 100000
D = 64
B = 4096
L = 50
LPAD = 56

NC, NS = 2, 16
NW = NC * NS
BPW = B // NW
LANES = 16
NBUF = 8
EPB = 2


def _sc_lookup_mean(idx_flat, table):
  mesh = plsc.VectorSubcoreMesh(core_axis_name="c", subcore_axis_name="s",
                                num_cores=NC, num_subcores=NS)

  @functools.partial(
      pl.kernel,
      out_type=jax.ShapeDtypeStruct((B, D), jnp.float32),
      mesh=mesh,
      compiler_params=pltpu.CompilerParams(use_tc_tiling_on_sc=False,
                                           needs_layout_passes=False),
      scratch_types=[
          pltpu.VMEM((BPW * LPAD,), jnp.int32),
          pltpu.VMEM((NBUF, EPB * LPAD, D), jnp.bfloat16),
          pltpu.VMEM((BPW, D), jnp.float32),
          [pltpu.SemaphoreType.DMA] * NBUF,
      ],
  )
  def k(idx_hbm, table_hbm, out_hbm, idx_v, rows_v, out_v, sems):
    wid = lax.axis_index("s") * NC + lax.axis_index("c")
    base = wid * BPW
    pltpu.sync_copy(idx_hbm.at[pl.ds(base * LPAD, BPW * LPAD)], idx_v)

    lane = lax.iota(jnp.int32, LANES)
    even_lane = lax.rem(lane, jnp.int32(2)) == 0
    dup_lo = lax.div(lane, jnp.int32(2))
    dup_hi = dup_lo + jnp.int32(LANES // 2)

    def interleave(a, b, sel):
      return jnp.where(even_lane,
                       a.at[sel].get(mode="promise_in_bounds"),
                       b.at[sel].get(mode="promise_in_bounds"))

    nblk = BPW // EPB

    def start(blk, j):
      pltpu.async_copy(
          table_hbm.at[idx_v.at[pl.ds(blk * EPB * LPAD, EPB * LPAD)]],
          rows_v.at[j], sems[j])

    for j in range(NBUF):
      start(j, j)

    def body(i, _):
      for j in range(NBUF):
        blk = i * NBUF + j
        pltpu.make_async_copy(
            table_hbm.at[idx_v.at[pl.ds(0, EPB * LPAD)]],
            rows_v.at[j], sems[j]).wait()
        for p in range(EPB):
          e = blk * EPB + p
          for g in range(D // (2 * LANES)):
            acc_lo = jnp.zeros((LANES,), jnp.float32)
            acc_hi = jnp.zeros((LANES,), jnp.float32)
            for r in range(L):
              chunk = rows_v[j, p * LPAD + r, pl.ds(g * 2 * LANES, 2 * LANES)]
              pair = plsc.bitcast(chunk, jnp.int32)
              acc_lo = acc_lo + plsc.bitcast(
                  lax.shift_left(pair, jnp.int32(16)), jnp.float32)
              acc_hi = acc_hi + plsc.bitcast(
                  lax.bitwise_and(pair, jnp.int32(-65536)), jnp.float32)
            acc_lo = acc_lo * jnp.float32(1.0 / L)
            acc_hi = acc_hi * jnp.float32(1.0 / L)
            out_v[e, pl.ds(g * 2 * LANES, LANES)] = interleave(
                acc_lo, acc_hi, dup_lo)
            out_v[e, pl.ds(g * 2 * LANES + LANES, LANES)] = interleave(
                acc_lo, acc_hi, dup_hi)
        start(jnp.minimum(blk + NBUF, nblk - 1), j)
      return 0

    lax.fori_loop(0, nblk // NBUF, body, 0)
    for j in range(NBUF):
      pltpu.make_async_copy(
          table_hbm.at[idx_v.at[pl.ds(0, EPB * LPAD)]],
          rows_v.at[j], sems[j]).wait()
    pltpu.sync_copy(out_v, out_hbm.at[pl.ds(base, BPW)])

  return k(idx_flat, table)


def kernel(idx, embedding):
  idx32 = idx.astype(jnp.int32)
  idx_pad = jnp.concatenate([idx32, idx32[:, : LPAD - L]], axis=1).reshape(-1)
  out = _sc_lookup_mean(idx_pad, embedding.astype(jnp.bfloat16))
  return out[:, None, :]


# speedup vs baseline: 1.0827x; 1.0827x over previous
import functools

import jax
import jax.numpy as jnp
from jax import lax
from jax.experimental import pallas as pl
from jax.experimental.pallas import tpu as pltpu
from jax.experimental.pallas import tpu_sc as plsc

VOCAB = 100000
D = 64
B = 4096
L = 50
LPAD = 56

NC, NS = 2, 16
NW = NC * NS
BPW = B // NW
LANES = 16
NBUF = 4
EPB = 2


def _sc_lookup_mean(idx_flat, table):
  mesh = plsc.VectorSubcoreMesh(core_axis_name="c", subcore_axis_name="s",
                                num_cores=NC, num_subcores=NS)

  @functools.partial(
      pl.kernel,
      out_type=jax.ShapeDtypeStruct((B, D), jnp.float32),
      mesh=mesh,
      compiler_params=pltpu.CompilerParams(use_tc_tiling_on_sc=False,
                                           needs_layout_passes=False),
      scratch_types=[
          pltpu.VMEM((BPW * LPAD,), jnp.int32),
          pltpu.VMEM((NBUF, EPB * LPAD, D), jnp.bfloat16),
          pltpu.VMEM((BPW, D), jnp.float32),
          [pltpu.SemaphoreType.DMA] * NBUF,
      ],
  )
  def k(idx_hbm, table_hbm, out_hbm, idx_v, rows_v, out_v, sems):
    wid = lax.axis_index("s") * NC + lax.axis_index("c")
    base = wid * BPW
    pltpu.sync_copy(idx_hbm.at[pl.ds(base * LPAD, BPW * LPAD)], idx_v)

    lane = lax.iota(jnp.int32, LANES)
    even_lane = lax.rem(lane, jnp.int32(2)) == 0
    dup_lo = lax.div(lane, jnp.int32(2))
    dup_hi = dup_lo + jnp.int32(LANES // 2)

    def interleave(a, b, sel):
      return jnp.where(even_lane,
                       a.at[sel].get(mode="promise_in_bounds"),
                       b.at[sel].get(mode="promise_in_bounds"))

    nblk = BPW // EPB

    def start(blk, j):
      pltpu.async_copy(
          table_hbm.at[idx_v.at[pl.ds(blk * EPB * LPAD, EPB * LPAD)]],
          rows_v.at[j], sems[j])

    for j in range(NBUF):
      start(j, j)

    def body(i, _):
      for j in range(NBUF):
        blk = i * NBUF + j
        pltpu.make_async_copy(
            table_hbm.at[idx_v.at[pl.ds(0, EPB * LPAD)]],
            rows_v.at[j], sems[j]).wait()
        for p in range(EPB):
          e = blk * EPB + p
          for g in range(D // (2 * LANES)):
            acc_lo = jnp.zeros((LANES,), jnp.float32)
            acc_hi = jnp.zeros((LANES,), jnp.float32)
            for r in range(L):
              chunk = rows_v[j, p * LPAD + r, pl.ds(g * 2 * LANES, 2 * LANES)]
              pair = plsc.bitcast(chunk, jnp.int32)
              acc_lo = acc_lo + plsc.bitcast(
                  lax.shift_left(pair, jnp.int32(16)), jnp.float32)
              acc_hi = acc_hi + plsc.bitcast(
                  lax.bitwise_and(pair, jnp.int32(-65536)), jnp.float32)
            acc_lo = acc_lo * jnp.float32(1.0 / L)
            acc_hi = acc_hi * jnp.float32(1.0 / L)
            out_v[e, pl.ds(g * 2 * LANES, LANES)] = interleave(
                acc_lo, acc_hi, dup_lo)
            out_v[e, pl.ds(g * 2 * LANES + LANES, LANES)] = interleave(
                acc_lo, acc_hi, dup_hi)
        start(jnp.minimum(blk + NBUF, nblk - 1), j)
      return 0

    lax.fori_loop(0, nblk // NBUF, body, 0)
    for j in range(NBUF):
      pltpu.make_async_copy(
          table_hbm.at[idx_v.at[pl.ds(0, EPB * LPAD)]],
          rows_v.at[j], sems[j]).wait()
    pltpu.sync_copy(out_v, out_hbm.at[pl.ds(base, BPW)])

  return k(idx_flat, table)


def kernel(idx, embedding):
  idx32 = idx.astype(jnp.int32)
  idx_pad = jnp.concatenate([idx32, idx32[:, : LPAD - L]], axis=1).reshape(-1)
  out = _sc_lookup_mean(idx_pad, embedding.astype(jnp.bfloat16))
  return out[:, None, :]
